# baseline (speedup 1.0000x reference)
import functools

import jax
import jax.numpy as jnp
from jax import lax
from jax.experimental import pallas as pl
from jax.experimental.pallas import tpu as pltpu
from jax.experimental.pallas import tpu_sc as plsc

SEQ = 577
NUM_HEADS = 12
NRD = 2212
ROW_PAD = 640
BATCH = 8


NR = 19
HGRP = 6
CHW = NR * ROW_PAD


PLANE = SEQ * ROW_PAD


def _gather_body(table_hbm, idx_hbm, out_hbm, table_v, idx_v, buf_v, osem):
    c = lax.axis_index("c")
    s = lax.axis_index("s")
    wid = s * 2 + c
    r0 = 18 * wid

    pltpu.sync_copy(table_hbm, table_v)
    pltpu.sync_copy(idx_hbm.at[pl.ds(r0 * ROW_PAD, CHW)], idx_v)

    for g in range(NUM_HEADS // HGRP):

        @plsc.parallel_loop(0, CHW, 16, unroll=2)
        def vec_loop(j):
            idxv = idx_v[pl.ds(j, 16)]
            for hl in range(HGRP):
                buf_v[pl.ds(hl * CHW + j, 16)] = plsc.load_gather(
                    table_v, [idxv + (g * HGRP + hl) * NRD])

        for hl in range(HGRP):
            pltpu.async_copy(buf_v.at[pl.ds(hl * CHW, CHW)],
                             out_hbm.at[pl.ds((g * HGRP + hl) * PLANE
                                              + r0 * ROW_PAD, CHW)],
                             osem)
        for hl in range(HGRP):
            pltpu.make_async_copy(buf_v.at[pl.ds(0, CHW)],
                                  out_hbm.at[pl.ds(0, CHW)], osem).wait()


@functools.cache
def _gather_call():
    return pl.kernel(
        _gather_body,
        out_type=jax.ShapeDtypeStruct((NUM_HEADS * SEQ * ROW_PAD,),
                                      jnp.float32),
        mesh=plsc.VectorSubcoreMesh(core_axis_name="c", subcore_axis_name="s"),
        scratch_types=[
            pltpu.VMEM((NUM_HEADS * NRD,), jnp.float32),
            pltpu.VMEM((CHW,), jnp.int32),
            pltpu.VMEM((HGRP * CHW,), jnp.float32),
            pltpu.SemaphoreType.DMA,
        ],
        compiler_params=pltpu.CompilerParams(needs_layout_passes=False),
    )


BB = 4


def _add_body(attn_ref, pos_ref, out_ref):
    bias = pos_ref[0][:, :SEQ]
    for b in range(BB):
        out_ref[b, 0] = attn_ref[b, 0] + bias


@functools.partial(jax.jit, donate_argnums=())
def _add_call(attn, pos):
    return pl.pallas_call(
        _add_body,
        grid=(NUM_HEADS, BATCH // BB),
        in_specs=[
            pl.BlockSpec((BB, 1, SEQ, SEQ), lambda h, b: (b, h, 0, 0)),
            pl.BlockSpec((1, SEQ, ROW_PAD), lambda h, b: (h, 0, 0)),
        ],
        out_specs=pl.BlockSpec((BB, 1, SEQ, SEQ), lambda h, b: (b, h, 0, 0)),
        out_shape=jax.ShapeDtypeStruct((BATCH, NUM_HEADS, SEQ, SEQ),
                                       jnp.float32),
    )(attn, pos)


def kernel(attention_scores, relative_position_bias_table,
           relative_position_index):
    table_t = relative_position_bias_table.T.reshape(-1)
    idx_pad = jnp.pad(relative_position_index,
                      ((0, 0), (0, ROW_PAD - SEQ))).reshape(-1)
    pos = _gather_call()(table_t, idx_pad)
    return _add_call(attention_scores,
                     pos.reshape(NUM_HEADS, SEQ, ROW_PAD))
